# baseline (speedup 1.0000x reference)
import jax
import jax.numpy as jnp
from jax.experimental import pallas as pl
from jax.experimental.pallas import tpu as pltpu


def _copy_body(src_ref, dst_ref, sem):
    copy = pltpu.make_async_copy(src_ref, dst_ref.at[0], sem)
    copy.start()
    copy.wait()


def kernel(x, embedding_weight):
    seq = x.shape[1]
    dim = embedding_weight.shape[1]
    return pl.pallas_call(
        _copy_body,
        out_shape=jax.ShapeDtypeStruct((1, seq, dim), embedding_weight.dtype),
        in_specs=[pl.BlockSpec(memory_space=pltpu.MemorySpace.HBM)],
        out_specs=pl.BlockSpec(memory_space=pltpu.MemorySpace.HBM),
        scratch_shapes=[pltpu.SemaphoreType.DMA],
    )(embedding_weight[:seq])
